# Initial kernel scaffold; baseline (speedup 1.0000x reference)
import jax
import jax.numpy as jnp
from jax.experimental import pallas as pl


def kernel(nodes, edges, starter, assignment, cursor_position, vars_in_scope, params):
    raise NotImplementedError("write your pallas kernel here")


# speedup vs baseline: 9.3800x; 9.3800x over previous
import jax
import jax.numpy as jnp
from jax.experimental import pallas as pl
from jax.experimental.pallas import tpu as pltpu

_B, _N, _E = 8, 1024, 4096
_EMBED = 512
_MAXV = 11
_NVOC = 58 + _MAXV * 2 + 1
_EVOC = (6 + 1) * 2
_OUTS = [128, 64, 64, 64, 64, 32]
_HEADS = [8, 8, 16, 1, 1, 1]
_INS = [_EMBED + 1, 128, 64, 64, 64, 64]
_HAS_SELF = [True, True, False, False, False, True]
_HAS_EDGE = [True, True, True, True, True, False]
_EDGES = 2 * _E
_EC = 512
_NCH = _EDGES // _EC
_ECR = _EC // 128
_ROWS = _EDGES // 128


def _onehot_rows(idx2d, width):
    r = idx2d.shape[0]
    iota = jax.lax.broadcasted_iota(jnp.int32, (r, 128, width), 2)
    oh = (idx2d[:, :, None] == iota).astype(jnp.float32)
    return oh.reshape(r * 128, width)


def _head_dot(m, att, H, C):
    cols = [jnp.sum(m[:, h * C:(h + 1) * C] * att[h:h + 1, :],
                    axis=1, keepdims=True) for h in range(H)]
    return cols[0] if H == 1 else jnp.concatenate(cols, axis=1)


def _embed_body(nodes_ref, starter_ref, emb_ref, o_ref):
    oh = _onehot_rows(nodes_ref[0] + 1, _NVOC)
    x = jnp.dot(oh, emb_ref[...], preferred_element_type=jnp.float32)
    o_ref[0] = jnp.concatenate([x, starter_ref[0]], axis=1)


def _spec(shape, imap):
    return pl.BlockSpec(shape, imap)


def _stage_a(i, x, p, eemb_tab):
    H, C, IN = _HEADS[i], _OUTS[i], _INS[i]
    HC = H * C
    has_e, has_s = _HAS_EDGE[i], _HAS_SELF[i]

    def body(*refs):
        it = iter(refs)
        x_ref = next(it)
        Wm_ref = next(it)
        bm_ref = next(it)
        if has_e:
            eemb_ref = next(it)
            We_ref = next(it)
            be_ref = next(it)
        att_ref = next(it)
        if has_s:
            Ws_ref = next(it)
            bs_ref = next(it)
        xm_ref = next(it)
        sx_ref = next(it)
        base_ref = next(it)
        if has_e:
            ee_ref = next(it)
            se_ref = next(it)

        x_ = x_ref[0]
        att = att_ref[...]
        xm = jnp.dot(x_, Wm_ref[...],
                     preferred_element_type=jnp.float32) + bm_ref[...]
        xm_ref[0] = xm
        sx_ref[0] = _head_dot(xm, att, H, C)
        if has_s:
            base_ref[0] = jnp.dot(x_, Ws_ref[...],
                                  preferred_element_type=jnp.float32) \
                + bs_ref[...]
        else:
            base_ref[0] = x_
        if has_e:
            ee = jnp.dot(eemb_ref[...], We_ref[...],
                         preferred_element_type=jnp.float32) + be_ref[...]
            ee_ref[...] = ee
            se_ref[...] = _head_dot(ee, att, H, C)

    in_specs = [_spec((1, _N, IN), lambda b: (b, 0, 0)),
                _spec((IN, HC), lambda b: (0, 0)),
                _spec((1, HC), lambda b: (0, 0))]
    args = [x, p['W_msg'], p['b_msg'].reshape(1, HC)]
    if has_e:
        in_specs += [_spec((_EVOC, _EMBED), lambda b: (0, 0)),
                     _spec((_EMBED, HC), lambda b: (0, 0)),
                     _spec((1, HC), lambda b: (0, 0))]
        args += [eemb_tab, p['W_edge'], p['b_edge'].reshape(1, HC)]
    in_specs.append(_spec((H, C), lambda b: (0, 0)))
    args.append(p['att_msg'])
    if has_s:
        in_specs += [_spec((IN, C), lambda b: (0, 0)),
                     _spec((1, C), lambda b: (0, 0))]
        args += [p['W_self'], p['b_self'].reshape(1, C)]

    out_specs = [_spec((1, _N, HC), lambda b: (b, 0, 0)),
                 _spec((1, _N, H), lambda b: (b, 0, 0)),
                 _spec((1, _N, C), lambda b: (b, 0, 0))]
    out_shape = [jax.ShapeDtypeStruct((_B, _N, HC), jnp.float32),
                 jax.ShapeDtypeStruct((_B, _N, H), jnp.float32),
                 jax.ShapeDtypeStruct((_B, _N, C), jnp.float32)]
    if has_e:
        out_specs += [_spec((_EVOC, HC), lambda b: (0, 0)),
                      _spec((_EVOC, H), lambda b: (0, 0))]
        out_shape += [jax.ShapeDtypeStruct((_EVOC, HC), jnp.float32),
                      jax.ShapeDtypeStruct((_EVOC, H), jnp.float32)]
    res = pl.pallas_call(body, grid=(_B,), in_specs=in_specs,
                         out_specs=out_specs, out_shape=out_shape)(*args)
    if has_e:
        return res
    return res + (None, None)


def _stage_b1(i, s_x, s_e, src2d, attr2d):
    H = _HEADS[i]
    has_e = _HAS_EDGE[i]

    def body(*refs):
        it = iter(refs)
        sx_ref = next(it)
        if has_e:
            se_ref = next(it)
        src_ref = next(it)
        if has_e:
            attr_ref = next(it)
        alpha_ref = next(it)
        gmax_ref = next(it)
        acc = next(it)

        k = pl.program_id(1)
        oh_s = _onehot_rows(src_ref[0, 0], _N)
        a = jnp.dot(oh_s, sx_ref[0], preferred_element_type=jnp.float32)
        if has_e:
            oh_a = _onehot_rows(attr_ref[0, 0], _EVOC)
            a = a + jnp.dot(oh_a, se_ref[...],
                            preferred_element_type=jnp.float32)
        a = jnp.where(a >= 0, a, 0.2 * a)
        alpha_ref[0] = a
        cmax = jnp.max(a, axis=0, keepdims=True)

        @pl.when(k == 0)
        def _():
            acc[...] = cmax

        @pl.when(k > 0)
        def _():
            acc[...] = jnp.maximum(acc[...], cmax)

        @pl.when(k == _NCH - 1)
        def _():
            gmax_ref[0] = acc[...]

    in_specs = [_spec((1, _N, H), lambda b, k: (b, 0, 0))]
    args = [s_x]
    if has_e:
        in_specs.append(_spec((_EVOC, H), lambda b, k: (0, 0)))
        args.append(s_e)
    in_specs.append(_spec((1, 1, _ECR, 128), lambda b, k: (b, k, 0, 0)))
    args.append(src2d)
    if has_e:
        in_specs.append(_spec((1, 1, _ECR, 128), lambda b, k: (b, k, 0, 0)))
        args.append(attr2d)
    return pl.pallas_call(
        body, grid=(_B, _NCH), in_specs=in_specs,
        out_specs=[_spec((1, _EC, H), lambda b, k: (b, k, 0)),
                   _spec((1, 1, H), lambda b, k: (b, 0, 0))],
        out_shape=[jax.ShapeDtypeStruct((_B, _EDGES, H), jnp.float32),
                   jax.ShapeDtypeStruct((_B, 1, H), jnp.float32)],
        scratch_shapes=[pltpu.VMEM((1, H), jnp.float32)],
    )(*args)


def _stage_b2(i, alpha, gmax, dstR):
    H = _HEADS[i]

    def body(alpha_ref, gmax_ref, dstR_ref, eall_ref, denom_ref, acc):
        k = pl.program_id(1)
        ec = jnp.exp(alpha_ref[0] - gmax_ref[0])
        eall_ref[0] = ec
        iota_ne = jax.lax.broadcasted_iota(jnp.int32, (_N, _EC), 0)
        ohT = (dstR_ref[0] == iota_ne).astype(jnp.float32)
        part = jnp.dot(ohT, ec, preferred_element_type=jnp.float32)

        @pl.when(k == 0)
        def _():
            acc[...] = part

        @pl.when(k > 0)
        def _():
            acc[...] = acc[...] + part

        @pl.when(k == _NCH - 1)
        def _():
            denom_ref[0] = acc[...]

    return pl.pallas_call(
        body, grid=(_B, _NCH),
        in_specs=[_spec((1, _EC, H), lambda b, k: (b, k, 0)),
                  _spec((1, 1, H), lambda b, k: (b, 0, 0)),
                  _spec((1, 1, _EC), lambda b, k: (b, 0, k))],
        out_specs=[_spec((1, _EC, H), lambda b, k: (b, k, 0)),
                   _spec((1, _N, H), lambda b, k: (b, 0, 0))],
        out_shape=[jax.ShapeDtypeStruct((_B, _EDGES, H), jnp.float32),
                   jax.ShapeDtypeStruct((_B, _N, H), jnp.float32)],
        scratch_shapes=[pltpu.VMEM((_N, H), jnp.float32)],
    )(alpha, gmax, dstR)


def _stage_b3(i, eall, denom, xm, ee, base, src2d, dst2d, dstR, attr2d):
    H, C = _HEADS[i], _OUTS[i]
    HC = H * C
    has_e = _HAS_EDGE[i]

    def body(*refs):
        it = iter(refs)
        eall_ref = next(it)
        denom_ref = next(it)
        xm_ref = next(it)
        if has_e:
            ee_ref = next(it)
        base_ref = next(it)
        src_ref = next(it)
        dst_ref = next(it)
        dstR_ref = next(it)
        if has_e:
            attr_ref = next(it)
        out_ref = next(it)
        acc = next(it)

        k = pl.program_id(1)
        oh_s = _onehot_rows(src_ref[0, 0], _N)
        g = jnp.dot(oh_s, xm_ref[0], preferred_element_type=jnp.float32)
        if has_e:
            oh_a = _onehot_rows(attr_ref[0, 0], _EVOC)
            g = g + jnp.dot(oh_a, ee_ref[...],
                            preferred_element_type=jnp.float32)
        oh_d = _onehot_rows(dst_ref[0, 0], _N)
        gd = jnp.dot(oh_d, denom_ref[0],
                     preferred_element_type=jnp.float32)
        aw = eall_ref[0] / (gd + 1e-16)
        if H == 1:
            w = g * aw
        else:
            w = jnp.concatenate(
                [g[:, h * C:(h + 1) * C] * aw[:, h:h + 1]
                 for h in range(H)], axis=1)
        iota_ne = jax.lax.broadcasted_iota(jnp.int32, (_N, _EC), 0)
        ohT = (dstR_ref[0] == iota_ne).astype(jnp.float32)
        part = jnp.dot(ohT, w, preferred_element_type=jnp.float32)

        @pl.when(k == 0)
        def _():
            acc[...] = part

        @pl.when(k > 0)
        def _():
            acc[...] = acc[...] + part

        @pl.when(k == _NCH - 1)
        def _():
            agg = acc[...]
            if H == 1:
                m = agg
            else:
                m = agg[:, 0:C]
                for h in range(1, H):
                    m = m + agg[:, h * C:(h + 1) * C]
                m = m * (1.0 / H)
            out = m + base_ref[0]
            if i < 5:
                out = jnp.where(out > 0,
                                out, jnp.exp(jnp.minimum(out, 0.0)) - 1.0)
            out_ref[0] = out

    in_specs = [_spec((1, _EC, H), lambda b, k: (b, k, 0)),
                _spec((1, _N, H), lambda b, k: (b, 0, 0)),
                _spec((1, _N, HC), lambda b, k: (b, 0, 0))]
    args = [eall, denom, xm]
    if has_e:
        in_specs.append(_spec((_EVOC, HC), lambda b, k: (0, 0)))
        args.append(ee)
    in_specs += [_spec((1, _N, C), lambda b, k: (b, 0, 0)),
                 _spec((1, 1, _ECR, 128), lambda b, k: (b, k, 0, 0)),
                 _spec((1, 1, _ECR, 128), lambda b, k: (b, k, 0, 0)),
                 _spec((1, 1, _EC), lambda b, k: (b, 0, k))]
    args += [base, src2d, dst2d, dstR]
    if has_e:
        in_specs.append(_spec((1, 1, _ECR, 128), lambda b, k: (b, k, 0, 0)))
        args.append(attr2d)
    return pl.pallas_call(
        body, grid=(_B, _NCH), in_specs=in_specs,
        out_specs=_spec((1, _N, C), lambda b, k: (b, 0, 0)),
        out_shape=jax.ShapeDtypeStruct((_B, _N, C), jnp.float32),
        scratch_shapes=[pltpu.VMEM((_N, HC), jnp.float32)],
    )(*args)


def _layer_call(i, x, src2d, dst2d, dstR, attr2d, p, eemb_tab):
    xm, s_x, base, ee, s_e = _stage_a(i, x, p, eemb_tab)
    alpha, gmax = _stage_b1(i, s_x, s_e, src2d, attr2d)
    eall, denom = _stage_b2(i, alpha, gmax, dstR)
    return _stage_b3(i, eall, denom, xm, ee, base,
                     src2d, dst2d, dstR, attr2d)


def _final_body(x_ref, cur_ref, vis_ref, Wc_ref, bc_ref,
                crit_ref, out_ref, vars_ref):
    x = x_ref[0]
    c = cur_ref[0]
    iota1n = jax.lax.broadcasted_iota(jnp.int32, (1, _N), 1)
    oh_c = (c == iota1n).astype(jnp.float32)
    outr = jnp.dot(oh_c, x, preferred_element_type=jnp.float32)
    crit = jnp.dot(outr, Wc_ref[...],
                   preferred_element_type=jnp.float32) + bc_ref[...]
    v = vis_ref[0]
    cnt = jnp.sum((v + 1 != 0).astype(jnp.int32))
    v2 = jnp.where(v < 0, v + _N, v)
    iotavn = jax.lax.broadcasted_iota(jnp.int32, (_MAXV, _N), 1)
    oh_v = (v2 == iotavn).astype(jnp.float32)
    vars_ = jnp.dot(oh_v, x, preferred_element_type=jnp.float32)
    jidx = jax.lax.broadcasted_iota(jnp.int32, (_MAXV, 1), 0)
    mask = jnp.where(cnt <= _MAXV - 1,
                     (jidx >= cnt).astype(jnp.float32), 0.0)
    vars_ = vars_ * (1.0 - mask)
    crit_ref[0] = crit
    out_ref[0] = outr
    vars_ref[0] = vars_


def _forward_impl(nodes, edges, starter, assignment, cursor_position,
                  vars_in_scope, params):
    del assignment
    nodes2d = nodes.astype(jnp.int32).reshape(_B, _N // 128, 128)
    starter3 = starter.astype(jnp.float32).reshape(_B, _N, 1)
    e3 = edges.astype(jnp.int32).reshape(_B, _E, 3)
    src, dst, ea = e3[:, :, 0], e3[:, :, 1], e3[:, :, 2]
    srcF = jnp.concatenate([src, dst], axis=1)
    dstF = jnp.concatenate([dst, src], axis=1)
    attrF = jnp.concatenate([ea + 1, ea + 7], axis=1)
    src2d = srcF.reshape(_B, _NCH, _ECR, 128)
    dst2d = dstF.reshape(_B, _NCH, _ECR, 128)
    attr2d = attrF.reshape(_B, _NCH, _ECR, 128)
    dstR = dstF.reshape(_B, 1, _EDGES)

    x = pl.pallas_call(
        _embed_body,
        grid=(_B,),
        in_specs=[_spec((1, _N // 128, 128), lambda b: (b, 0, 0)),
                  _spec((1, _N, 1), lambda b: (b, 0, 0)),
                  _spec((_NVOC, _EMBED), lambda b: (0, 0))],
        out_specs=_spec((1, _N, _INS[0]), lambda b: (b, 0, 0)),
        out_shape=jax.ShapeDtypeStruct((_B, _N, _INS[0]), jnp.float32),
    )(nodes2d, starter3, params['node_emb'])

    for i in range(6):
        x = _layer_call(i, x, src2d, dst2d, dstR, attr2d,
                        params['conv%d' % i], params['edge_emb'])

    cur3 = cursor_position.astype(jnp.int32).reshape(_B, 1, 1)
    vis3 = vars_in_scope.astype(jnp.int32).reshape(_B, _MAXV, 1)
    crit, out, vars_ = pl.pallas_call(
        _final_body,
        grid=(_B,),
        in_specs=[_spec((1, _N, _OUTS[5]), lambda b: (b, 0, 0)),
                  _spec((1, 1, 1), lambda b: (b, 0, 0)),
                  _spec((1, _MAXV, 1), lambda b: (b, 0, 0)),
                  _spec((_OUTS[5], 1), lambda b: (0, 0)),
                  _spec((1, 1), lambda b: (0, 0))],
        out_specs=[_spec((1, 1, 1), lambda b: (b, 0, 0)),
                   _spec((1, 1, _OUTS[5]), lambda b: (b, 0, 0)),
                   _spec((1, _MAXV, _OUTS[5]), lambda b: (b, 0, 0))],
        out_shape=[jax.ShapeDtypeStruct((_B, 1, 1), jnp.float32),
                   jax.ShapeDtypeStruct((_B, 1, _OUTS[5]), jnp.float32),
                   jax.ShapeDtypeStruct((_B, _MAXV, _OUTS[5]), jnp.float32)],
    )(x, cur3, vis3, params['critic_W'], params['critic_b'].reshape(1, 1))
    return (crit.reshape(_B, 1), out.reshape(_B, _OUTS[5]), vars_)


kernel = jax.jit(_forward_impl)


# speedup vs baseline: 11.5884x; 1.2354x over previous
import jax
import jax.numpy as jnp
from jax.experimental import pallas as pl
from jax.experimental.pallas import tpu as pltpu

_B, _N, _E = 8, 1024, 4096
_EMBED = 512
_MAXV = 11
_NVOC = 58 + _MAXV * 2 + 1
_EVOC = (6 + 1) * 2
_OUTS = [128, 64, 64, 64, 64, 32]
_HEADS = [8, 8, 16, 1, 1, 1]
_INS = [_EMBED + 1, 128, 64, 64, 64, 64]
_HAS_SELF = [True, True, False, False, False, True]
_HAS_EDGE = [True, True, True, True, True, False]
_EDGES = 2 * _E
_EC = 1024
_NCH = _EDGES // _EC
_ECR = _EC // 128
_ROWS = _EDGES // 128


def _onehot_rows(idx2d, width, dtype=jnp.float32):
    r = idx2d.shape[0]
    iota = jax.lax.broadcasted_iota(jnp.int32, (r, 128, width), 2)
    oh = (idx2d[:, :, None] == iota).astype(dtype)
    return oh.reshape(r * 128, width)


def _head_dot(m, att, H, C):
    cols = [jnp.sum(m[:, h * C:(h + 1) * C] * att[h:h + 1, :],
                    axis=1, keepdims=True) for h in range(H)]
    return cols[0] if H == 1 else jnp.concatenate(cols, axis=1)


def _embed_body(nodes_ref, starter_ref, emb_ref, o_ref):
    oh = _onehot_rows(nodes_ref[0] + 1, _NVOC)
    x = jnp.dot(oh, emb_ref[...], preferred_element_type=jnp.float32)
    o_ref[0] = jnp.concatenate([x, starter_ref[0]], axis=1)


def _spec(shape, imap):
    return pl.BlockSpec(shape, imap)


def _stage_a(i, x, p, eemb_tab):
    H, C, IN = _HEADS[i], _OUTS[i], _INS[i]
    HC = H * C
    has_e, has_s = _HAS_EDGE[i], _HAS_SELF[i]

    def body(*refs):
        it = iter(refs)
        x_ref = next(it)
        Wm_ref = next(it)
        bm_ref = next(it)
        if has_e:
            eemb_ref = next(it)
            We_ref = next(it)
            be_ref = next(it)
        att_ref = next(it)
        if has_s:
            Ws_ref = next(it)
            bs_ref = next(it)
        xm_ref = next(it)
        sx_ref = next(it)
        base_ref = next(it)
        if has_e:
            ee_ref = next(it)
            se_ref = next(it)

        x_ = x_ref[0]
        att = att_ref[...]
        xm = jnp.dot(x_, Wm_ref[...],
                     preferred_element_type=jnp.float32) + bm_ref[...]
        xm_ref[0] = xm
        sx_ref[0] = _head_dot(xm, att, H, C)
        if has_s:
            base_ref[0] = jnp.dot(x_, Ws_ref[...],
                                  preferred_element_type=jnp.float32) \
                + bs_ref[...]
        else:
            base_ref[0] = x_
        if has_e:
            ee = jnp.dot(eemb_ref[...], We_ref[...],
                         preferred_element_type=jnp.float32) + be_ref[...]
            ee_ref[...] = ee
            se_ref[...] = _head_dot(ee, att, H, C)

    in_specs = [_spec((1, _N, IN), lambda b: (b, 0, 0)),
                _spec((IN, HC), lambda b: (0, 0)),
                _spec((1, HC), lambda b: (0, 0))]
    args = [x, p['W_msg'], p['b_msg'].reshape(1, HC)]
    if has_e:
        in_specs += [_spec((_EVOC, _EMBED), lambda b: (0, 0)),
                     _spec((_EMBED, HC), lambda b: (0, 0)),
                     _spec((1, HC), lambda b: (0, 0))]
        args += [eemb_tab, p['W_edge'], p['b_edge'].reshape(1, HC)]
    in_specs.append(_spec((H, C), lambda b: (0, 0)))
    args.append(p['att_msg'])
    if has_s:
        in_specs += [_spec((IN, C), lambda b: (0, 0)),
                     _spec((1, C), lambda b: (0, 0))]
        args += [p['W_self'], p['b_self'].reshape(1, C)]

    out_specs = [_spec((1, _N, HC), lambda b: (b, 0, 0)),
                 _spec((1, _N, H), lambda b: (b, 0, 0)),
                 _spec((1, _N, C), lambda b: (b, 0, 0))]
    out_shape = [jax.ShapeDtypeStruct((_B, _N, HC), jnp.float32),
                 jax.ShapeDtypeStruct((_B, _N, H), jnp.float32),
                 jax.ShapeDtypeStruct((_B, _N, C), jnp.float32)]
    if has_e:
        out_specs += [_spec((_EVOC, HC), lambda b: (0, 0)),
                      _spec((_EVOC, H), lambda b: (0, 0))]
        out_shape += [jax.ShapeDtypeStruct((_EVOC, HC), jnp.float32),
                      jax.ShapeDtypeStruct((_EVOC, H), jnp.float32)]
    res = pl.pallas_call(body, grid=(_B,), in_specs=in_specs,
                         out_specs=out_specs, out_shape=out_shape)(*args)
    if has_e:
        return res
    return res + (None, None)


def _stage_b1(i, s_x, s_e, src2d, attr2d):
    H = _HEADS[i]
    has_e = _HAS_EDGE[i]

    def body(*refs):
        it = iter(refs)
        sx_ref = next(it)
        if has_e:
            se_ref = next(it)
        src_ref = next(it)
        if has_e:
            attr_ref = next(it)
        alpha_ref = next(it)
        gmax_ref = next(it)
        acc = next(it)

        k = pl.program_id(1)
        oh_s = _onehot_rows(src_ref[0, 0], _N)
        a = jnp.dot(oh_s, sx_ref[0], preferred_element_type=jnp.float32)
        if has_e:
            oh_a = _onehot_rows(attr_ref[0, 0], _EVOC)
            a = a + jnp.dot(oh_a, se_ref[...],
                            preferred_element_type=jnp.float32)
        a = jnp.where(a >= 0, a, 0.2 * a)
        alpha_ref[0] = a
        cmax = jnp.max(a, axis=0, keepdims=True)

        @pl.when(k == 0)
        def _():
            acc[...] = cmax

        @pl.when(k > 0)
        def _():
            acc[...] = jnp.maximum(acc[...], cmax)

        @pl.when(k == _NCH - 1)
        def _():
            gmax_ref[0] = acc[...]

    in_specs = [_spec((1, _N, H), lambda b, k: (b, 0, 0))]
    args = [s_x]
    if has_e:
        in_specs.append(_spec((_EVOC, H), lambda b, k: (0, 0)))
        args.append(s_e)
    in_specs.append(_spec((1, 1, _ECR, 128), lambda b, k: (b, k, 0, 0)))
    args.append(src2d)
    if has_e:
        in_specs.append(_spec((1, 1, _ECR, 128), lambda b, k: (b, k, 0, 0)))
        args.append(attr2d)
    return pl.pallas_call(
        body, grid=(_B, _NCH), in_specs=in_specs,
        out_specs=[_spec((1, _EC, H), lambda b, k: (b, k, 0)),
                   _spec((1, 1, H), lambda b, k: (b, 0, 0))],
        out_shape=[jax.ShapeDtypeStruct((_B, _EDGES, H), jnp.float32),
                   jax.ShapeDtypeStruct((_B, 1, H), jnp.float32)],
        scratch_shapes=[pltpu.VMEM((1, H), jnp.float32)],
    )(*args)


def _stage_b2(i, alpha, gmax, dstR):
    H = _HEADS[i]

    def body(alpha_ref, gmax_ref, dstR_ref, eall_ref, denom_ref, acc):
        k = pl.program_id(1)
        ec = jnp.exp(alpha_ref[0] - gmax_ref[0])
        eall_ref[0] = ec
        iota_ne = jax.lax.broadcasted_iota(jnp.int32, (_N, _EC), 0)
        ohT = (dstR_ref[0] == iota_ne).astype(jnp.float32)
        part = jnp.dot(ohT, ec, preferred_element_type=jnp.float32)

        @pl.when(k == 0)
        def _():
            acc[...] = part

        @pl.when(k > 0)
        def _():
            acc[...] = acc[...] + part

        @pl.when(k == _NCH - 1)
        def _():
            denom_ref[0] = acc[...]

    return pl.pallas_call(
        body, grid=(_B, _NCH),
        in_specs=[_spec((1, _EC, H), lambda b, k: (b, k, 0)),
                  _spec((1, 1, H), lambda b, k: (b, 0, 0)),
                  _spec((1, 1, _EC), lambda b, k: (b, 0, k))],
        out_specs=[_spec((1, _EC, H), lambda b, k: (b, k, 0)),
                   _spec((1, _N, H), lambda b, k: (b, 0, 0))],
        out_shape=[jax.ShapeDtypeStruct((_B, _EDGES, H), jnp.float32),
                   jax.ShapeDtypeStruct((_B, _N, H), jnp.float32)],
        scratch_shapes=[pltpu.VMEM((_N, H), jnp.float32)],
    )(alpha, gmax, dstR)


def _stage_b3(i, eall, denom, xm, ee, base, src2d, dst2d, dstR, attr2d):
    H, C = _HEADS[i], _OUTS[i]
    HC = H * C
    has_e = _HAS_EDGE[i]

    def body(*refs):
        it = iter(refs)
        eall_ref = next(it)
        denom_ref = next(it)
        xm_ref = next(it)
        if has_e:
            ee_ref = next(it)
        base_ref = next(it)
        src_ref = next(it)
        dst_ref = next(it)
        dstR_ref = next(it)
        if has_e:
            attr_ref = next(it)
        out_ref = next(it)
        acc = next(it)

        k = pl.program_id(1)
        oh_s = _onehot_rows(src_ref[0, 0], _N, jnp.bfloat16)
        g = jnp.dot(oh_s, xm_ref[0].astype(jnp.bfloat16),
                    preferred_element_type=jnp.float32)
        if has_e:
            oh_a = _onehot_rows(attr_ref[0, 0], _EVOC)
            g = g + jnp.dot(oh_a, ee_ref[...],
                            preferred_element_type=jnp.float32)
        oh_d = _onehot_rows(dst_ref[0, 0], _N)
        gd = jnp.dot(oh_d, denom_ref[0],
                     preferred_element_type=jnp.float32)
        aw = eall_ref[0] / (gd + 1e-16)
        if H == 1:
            w = g * aw
        else:
            w = jnp.concatenate(
                [g[:, h * C:(h + 1) * C] * aw[:, h:h + 1]
                 for h in range(H)], axis=1)
        iota_ne = jax.lax.broadcasted_iota(jnp.int32, (_N, _EC), 0)
        ohT = (dstR_ref[0] == iota_ne).astype(jnp.bfloat16)
        part = jnp.dot(ohT, w.astype(jnp.bfloat16),
                       preferred_element_type=jnp.float32)

        @pl.when(k == 0)
        def _():
            acc[...] = part

        @pl.when(k > 0)
        def _():
            acc[...] = acc[...] + part

        @pl.when(k == _NCH - 1)
        def _():
            agg = acc[...]
            if H == 1:
                m = agg
            else:
                m = agg[:, 0:C]
                for h in range(1, H):
                    m = m + agg[:, h * C:(h + 1) * C]
                m = m * (1.0 / H)
            out = m + base_ref[0]
            if i < 5:
                out = jnp.where(out > 0,
                                out, jnp.exp(jnp.minimum(out, 0.0)) - 1.0)
            out_ref[0] = out

    in_specs = [_spec((1, _EC, H), lambda b, k: (b, k, 0)),
                _spec((1, _N, H), lambda b, k: (b, 0, 0)),
                _spec((1, _N, HC), lambda b, k: (b, 0, 0))]
    args = [eall, denom, xm]
    if has_e:
        in_specs.append(_spec((_EVOC, HC), lambda b, k: (0, 0)))
        args.append(ee)
    in_specs += [_spec((1, _N, C), lambda b, k: (b, 0, 0)),
                 _spec((1, 1, _ECR, 128), lambda b, k: (b, k, 0, 0)),
                 _spec((1, 1, _ECR, 128), lambda b, k: (b, k, 0, 0)),
                 _spec((1, 1, _EC), lambda b, k: (b, 0, k))]
    args += [base, src2d, dst2d, dstR]
    if has_e:
        in_specs.append(_spec((1, 1, _ECR, 128), lambda b, k: (b, k, 0, 0)))
        args.append(attr2d)
    return pl.pallas_call(
        body, grid=(_B, _NCH), in_specs=in_specs,
        out_specs=_spec((1, _N, C), lambda b, k: (b, 0, 0)),
        out_shape=jax.ShapeDtypeStruct((_B, _N, C), jnp.float32),
        scratch_shapes=[pltpu.VMEM((_N, HC), jnp.float32)],
    )(*args)


def _layer_call(i, x, src2d, dst2d, dstR, attr2d, p, eemb_tab):
    xm, s_x, base, ee, s_e = _stage_a(i, x, p, eemb_tab)
    alpha, gmax = _stage_b1(i, s_x, s_e, src2d, attr2d)
    eall, denom = _stage_b2(i, alpha, gmax, dstR)
    return _stage_b3(i, eall, denom, xm, ee, base,
                     src2d, dst2d, dstR, attr2d)


def _final_body(x_ref, cur_ref, vis_ref, Wc_ref, bc_ref,
                crit_ref, out_ref, vars_ref):
    x = x_ref[0]
    c = cur_ref[0]
    iota1n = jax.lax.broadcasted_iota(jnp.int32, (1, _N), 1)
    oh_c = (c == iota1n).astype(jnp.float32)
    outr = jnp.dot(oh_c, x, preferred_element_type=jnp.float32)
    crit = jnp.dot(outr, Wc_ref[...],
                   preferred_element_type=jnp.float32) + bc_ref[...]
    v = vis_ref[0]
    cnt = jnp.sum((v + 1 != 0).astype(jnp.int32))
    v2 = jnp.where(v < 0, v + _N, v)
    iotavn = jax.lax.broadcasted_iota(jnp.int32, (_MAXV, _N), 1)
    oh_v = (v2 == iotavn).astype(jnp.float32)
    vars_ = jnp.dot(oh_v, x, preferred_element_type=jnp.float32)
    jidx = jax.lax.broadcasted_iota(jnp.int32, (_MAXV, 1), 0)
    mask = jnp.where(cnt <= _MAXV - 1,
                     (jidx >= cnt).astype(jnp.float32), 0.0)
    vars_ = vars_ * (1.0 - mask)
    crit_ref[0] = crit
    out_ref[0] = outr
    vars_ref[0] = vars_


def _forward_impl(nodes, edges, starter, assignment, cursor_position,
                  vars_in_scope, params):
    del assignment
    nodes2d = nodes.astype(jnp.int32).reshape(_B, _N // 128, 128)
    starter3 = starter.astype(jnp.float32).reshape(_B, _N, 1)
    e3 = edges.astype(jnp.int32).reshape(_B, _E, 3)
    src, dst, ea = e3[:, :, 0], e3[:, :, 1], e3[:, :, 2]
    srcF = jnp.concatenate([src, dst], axis=1)
    dstF = jnp.concatenate([dst, src], axis=1)
    attrF = jnp.concatenate([ea + 1, ea + 7], axis=1)
    src2d = srcF.reshape(_B, _NCH, _ECR, 128)
    dst2d = dstF.reshape(_B, _NCH, _ECR, 128)
    attr2d = attrF.reshape(_B, _NCH, _ECR, 128)
    dstR = dstF.reshape(_B, 1, _EDGES)

    x = pl.pallas_call(
        _embed_body,
        grid=(_B,),
        in_specs=[_spec((1, _N // 128, 128), lambda b: (b, 0, 0)),
                  _spec((1, _N, 1), lambda b: (b, 0, 0)),
                  _spec((_NVOC, _EMBED), lambda b: (0, 0))],
        out_specs=_spec((1, _N, _INS[0]), lambda b: (b, 0, 0)),
        out_shape=jax.ShapeDtypeStruct((_B, _N, _INS[0]), jnp.float32),
    )(nodes2d, starter3, params['node_emb'])

    for i in range(6):
        x = _layer_call(i, x, src2d, dst2d, dstR, attr2d,
                        params['conv%d' % i], params['edge_emb'])

    cur3 = cursor_position.astype(jnp.int32).reshape(_B, 1, 1)
    vis3 = vars_in_scope.astype(jnp.int32).reshape(_B, _MAXV, 1)
    crit, out, vars_ = pl.pallas_call(
        _final_body,
        grid=(_B,),
        in_specs=[_spec((1, _N, _OUTS[5]), lambda b: (b, 0, 0)),
                  _spec((1, 1, 1), lambda b: (b, 0, 0)),
                  _spec((1, _MAXV, 1), lambda b: (b, 0, 0)),
                  _spec((_OUTS[5], 1), lambda b: (0, 0)),
                  _spec((1, 1), lambda b: (0, 0))],
        out_specs=[_spec((1, 1, 1), lambda b: (b, 0, 0)),
                   _spec((1, 1, _OUTS[5]), lambda b: (b, 0, 0)),
                   _spec((1, _MAXV, _OUTS[5]), lambda b: (b, 0, 0))],
        out_shape=[jax.ShapeDtypeStruct((_B, 1, 1), jnp.float32),
                   jax.ShapeDtypeStruct((_B, 1, _OUTS[5]), jnp.float32),
                   jax.ShapeDtypeStruct((_B, _MAXV, _OUTS[5]), jnp.float32)],
    )(x, cur3, vis3, params['critic_W'], params['critic_b'].reshape(1, 1))
    return (crit.reshape(_B, 1), out.reshape(_B, _OUTS[5]), vars_)


kernel = jax.jit(_forward_impl)


# speedup vs baseline: 14.7540x; 1.2732x over previous
import jax
import jax.numpy as jnp
from jax.experimental import pallas as pl
from jax.experimental.pallas import tpu as pltpu

_B, _N, _E = 8, 1024, 4096
_EMBED = 512
_MAXV = 11
_NVOC = 58 + _MAXV * 2 + 1
_EVOC = (6 + 1) * 2
_OUTS = [128, 64, 64, 64, 64, 32]
_HEADS = [8, 8, 16, 1, 1, 1]
_INS = [_EMBED + 1, 128, 64, 64, 64, 64]
_HAS_SELF = [True, True, False, False, False, True]
_HAS_EDGE = [True, True, True, True, True, False]
_EDGES = 2 * _E
_EC = 1024
_NCH = _EDGES // _EC
_ECR = _EC // 128
_ROWS = _EDGES // 128


def _onehot_rows(idx2d, width, dtype=jnp.float32):
    r = idx2d.shape[0]
    iota = jax.lax.broadcasted_iota(jnp.int32, (r, 128, width), 2)
    oh = (idx2d[:, :, None] == iota).astype(dtype)
    return oh.reshape(r * 128, width)


def _head_dot(m, att, H, C):
    cols = [jnp.sum(m[:, h * C:(h + 1) * C] * att[h:h + 1, :],
                    axis=1, keepdims=True) for h in range(H)]
    return cols[0] if H == 1 else jnp.concatenate(cols, axis=1)


def _embed_body(nodes_ref, starter_ref, emb_ref, o_ref):
    oh = _onehot_rows(nodes_ref[0] + 1, _NVOC)
    x = jnp.dot(oh, emb_ref[...], preferred_element_type=jnp.float32)
    o_ref[0] = jnp.concatenate([x, starter_ref[0]], axis=1)


def _spec(shape, imap):
    return pl.BlockSpec(shape, imap)


def _stage_a(i, x, p, eemb_tab):
    H, C, IN = _HEADS[i], _OUTS[i], _INS[i]
    HC = H * C
    has_e, has_s = _HAS_EDGE[i], _HAS_SELF[i]

    def body(*refs):
        it = iter(refs)
        x_ref = next(it)
        Wm_ref = next(it)
        bm_ref = next(it)
        if has_e:
            eemb_ref = next(it)
            We_ref = next(it)
            be_ref = next(it)
        att_ref = next(it)
        if has_s:
            Ws_ref = next(it)
            bs_ref = next(it)
        xm_ref = next(it)
        sx_ref = next(it)
        base_ref = next(it)
        if has_e:
            ee_ref = next(it)
            se_ref = next(it)

        x_ = x_ref[0]
        att = att_ref[...]
        xm = jnp.dot(x_, Wm_ref[...],
                     preferred_element_type=jnp.float32) + bm_ref[...]
        xm_ref[0] = xm
        sx_ref[0] = _head_dot(xm, att, H, C)
        if has_s:
            base_ref[0] = jnp.dot(x_, Ws_ref[...],
                                  preferred_element_type=jnp.float32) \
                + bs_ref[...]
        else:
            base_ref[0] = x_
        if has_e:
            ee = jnp.dot(eemb_ref[...], We_ref[...],
                         preferred_element_type=jnp.float32) + be_ref[...]
            ee_ref[...] = ee
            se_ref[...] = _head_dot(ee, att, H, C)

    in_specs = [_spec((1, _N, IN), lambda b: (b, 0, 0)),
                _spec((IN, HC), lambda b: (0, 0)),
                _spec((1, HC), lambda b: (0, 0))]
    args = [x, p['W_msg'], p['b_msg'].reshape(1, HC)]
    if has_e:
        in_specs += [_spec((_EVOC, _EMBED), lambda b: (0, 0)),
                     _spec((_EMBED, HC), lambda b: (0, 0)),
                     _spec((1, HC), lambda b: (0, 0))]
        args += [eemb_tab, p['W_edge'], p['b_edge'].reshape(1, HC)]
    in_specs.append(_spec((H, C), lambda b: (0, 0)))
    args.append(p['att_msg'])
    if has_s:
        in_specs += [_spec((IN, C), lambda b: (0, 0)),
                     _spec((1, C), lambda b: (0, 0))]
        args += [p['W_self'], p['b_self'].reshape(1, C)]

    out_specs = [_spec((1, _N, HC), lambda b: (b, 0, 0)),
                 _spec((1, _N, H), lambda b: (b, 0, 0)),
                 _spec((1, _N, C), lambda b: (b, 0, 0))]
    out_shape = [jax.ShapeDtypeStruct((_B, _N, HC), jnp.float32),
                 jax.ShapeDtypeStruct((_B, _N, H), jnp.float32),
                 jax.ShapeDtypeStruct((_B, _N, C), jnp.float32)]
    if has_e:
        out_specs += [_spec((_EVOC, HC), lambda b: (0, 0)),
                      _spec((_EVOC, H), lambda b: (0, 0))]
        out_shape += [jax.ShapeDtypeStruct((_EVOC, HC), jnp.float32),
                      jax.ShapeDtypeStruct((_EVOC, H), jnp.float32)]
    res = pl.pallas_call(body, grid=(_B,), in_specs=in_specs,
                         out_specs=out_specs, out_shape=out_shape)(*args)
    if has_e:
        return res
    return res + (None, None)


def _stage_b1(i, s_x, s_e, src2d, attr2d):
    H = _HEADS[i]
    has_e = _HAS_EDGE[i]

    def body(*refs):
        it = iter(refs)
        sx_ref = next(it)
        if has_e:
            se_ref = next(it)
        src_ref = next(it)
        if has_e:
            attr_ref = next(it)
        alpha_ref = next(it)
        gmax_ref = next(it)
        acc = next(it)

        k = pl.program_id(1)
        oh_s = _onehot_rows(src_ref[0, 0], _N)
        a = jnp.dot(oh_s, sx_ref[0], preferred_element_type=jnp.float32)
        if has_e:
            oh_a = _onehot_rows(attr_ref[0, 0], _EVOC)
            a = a + jnp.dot(oh_a, se_ref[...],
                            preferred_element_type=jnp.float32)
        a = jnp.where(a >= 0, a, 0.2 * a)
        alpha_ref[0] = a
        cmax = jnp.max(a, axis=0, keepdims=True)

        @pl.when(k == 0)
        def _():
            acc[...] = cmax

        @pl.when(k > 0)
        def _():
            acc[...] = jnp.maximum(acc[...], cmax)

        @pl.when(k == _NCH - 1)
        def _():
            gmax_ref[0] = acc[...]

    in_specs = [_spec((1, _N, H), lambda b, k: (b, 0, 0))]
    args = [s_x]
    if has_e:
        in_specs.append(_spec((_EVOC, H), lambda b, k: (0, 0)))
        args.append(s_e)
    in_specs.append(_spec((1, 1, _ECR, 128), lambda b, k: (b, k, 0, 0)))
    args.append(src2d)
    if has_e:
        in_specs.append(_spec((1, 1, _ECR, 128), lambda b, k: (b, k, 0, 0)))
        args.append(attr2d)
    return pl.pallas_call(
        body, grid=(_B, _NCH), in_specs=in_specs,
        out_specs=[_spec((1, _EC, H), lambda b, k: (b, k, 0)),
                   _spec((1, 1, H), lambda b, k: (b, 0, 0))],
        out_shape=[jax.ShapeDtypeStruct((_B, _EDGES, H), jnp.float32),
                   jax.ShapeDtypeStruct((_B, 1, H), jnp.float32)],
        scratch_shapes=[pltpu.VMEM((1, H), jnp.float32)],
    )(*args)


def _stage_b3(i, alpha, gmax, xm, ee, base, src2d, dstR, attr2d):
    H, C = _HEADS[i], _OUTS[i]
    HC = H * C
    has_e = _HAS_EDGE[i]

    def body(*refs):
        it = iter(refs)
        alpha_ref = next(it)
        gmax_ref = next(it)
        xm_ref = next(it)
        if has_e:
            ee_ref = next(it)
        base_ref = next(it)
        src_ref = next(it)
        dstR_ref = next(it)
        if has_e:
            attr_ref = next(it)
        out_ref = next(it)
        acc = next(it)

        k = pl.program_id(1)
        ec = jnp.exp(alpha_ref[0] - gmax_ref[0])
        oh_s = _onehot_rows(src_ref[0, 0], _N, jnp.bfloat16)
        g = jnp.dot(oh_s, xm_ref[0].astype(jnp.bfloat16),
                    preferred_element_type=jnp.float32)
        if has_e:
            oh_a = _onehot_rows(attr_ref[0, 0], _EVOC)
            g = g + jnp.dot(oh_a, ee_ref[...],
                            preferred_element_type=jnp.float32)
        if H == 1:
            w = g * ec
        else:
            w = jnp.concatenate(
                [g[:, h * C:(h + 1) * C] * ec[:, h:h + 1]
                 for h in range(H)], axis=1)
        w_aug = jnp.concatenate([w, ec], axis=1)
        iota_ne = jax.lax.broadcasted_iota(jnp.int32, (_N, _EC), 0)
        ohT = (dstR_ref[0] == iota_ne).astype(jnp.bfloat16)
        part = jnp.dot(ohT, w_aug.astype(jnp.bfloat16),
                       preferred_element_type=jnp.float32)

        @pl.when(k == 0)
        def _():
            acc[...] = part

        @pl.when(k > 0)
        def _():
            acc[...] = acc[...] + part

        @pl.when(k == _NCH - 1)
        def _():
            a_ = acc[...]
            if H == 1:
                m = a_[:, 0:C] / (a_[:, HC:HC + 1] + 1e-16)
            else:
                m = a_[:, 0:C] / (a_[:, HC:HC + 1] + 1e-16)
                for h in range(1, H):
                    m = m + a_[:, h * C:(h + 1) * C] \
                        / (a_[:, HC + h:HC + h + 1] + 1e-16)
                m = m * (1.0 / H)
            out = m + base_ref[0]
            if i < 5:
                out = jnp.where(out > 0,
                                out, jnp.exp(jnp.minimum(out, 0.0)) - 1.0)
            out_ref[0] = out

    in_specs = [_spec((1, _EC, H), lambda b, k: (b, k, 0)),
                _spec((1, 1, H), lambda b, k: (b, 0, 0)),
                _spec((1, _N, HC), lambda b, k: (b, 0, 0))]
    args = [alpha, gmax, xm]
    if has_e:
        in_specs.append(_spec((_EVOC, HC), lambda b, k: (0, 0)))
        args.append(ee)
    in_specs += [_spec((1, _N, C), lambda b, k: (b, 0, 0)),
                 _spec((1, 1, _ECR, 128), lambda b, k: (b, k, 0, 0)),
                 _spec((1, 1, _EC), lambda b, k: (b, 0, k))]
    args += [base, src2d, dstR]
    if has_e:
        in_specs.append(_spec((1, 1, _ECR, 128), lambda b, k: (b, k, 0, 0)))
        args.append(attr2d)
    return pl.pallas_call(
        body, grid=(_B, _NCH), in_specs=in_specs,
        out_specs=_spec((1, _N, C), lambda b, k: (b, 0, 0)),
        out_shape=jax.ShapeDtypeStruct((_B, _N, C), jnp.float32),
        scratch_shapes=[pltpu.VMEM((_N, HC + H), jnp.float32)],
    )(*args)


def _layer_call(i, x, src2d, dst2d, dstR, attr2d, p, eemb_tab):
    xm, s_x, base, ee, s_e = _stage_a(i, x, p, eemb_tab)
    alpha, gmax = _stage_b1(i, s_x, s_e, src2d, attr2d)
    return _stage_b3(i, alpha, gmax, xm, ee, base, src2d, dstR, attr2d)


def _final_body(x_ref, cur_ref, vis_ref, Wc_ref, bc_ref,
                crit_ref, out_ref, vars_ref):
    x = x_ref[0]
    c = cur_ref[0]
    iota1n = jax.lax.broadcasted_iota(jnp.int32, (1, _N), 1)
    oh_c = (c == iota1n).astype(jnp.float32)
    outr = jnp.dot(oh_c, x, preferred_element_type=jnp.float32)
    crit = jnp.dot(outr, Wc_ref[...],
                   preferred_element_type=jnp.float32) + bc_ref[...]
    v = vis_ref[0]
    cnt = jnp.sum((v + 1 != 0).astype(jnp.int32))
    v2 = jnp.where(v < 0, v + _N, v)
    iotavn = jax.lax.broadcasted_iota(jnp.int32, (_MAXV, _N), 1)
    oh_v = (v2 == iotavn).astype(jnp.float32)
    vars_ = jnp.dot(oh_v, x, preferred_element_type=jnp.float32)
    jidx = jax.lax.broadcasted_iota(jnp.int32, (_MAXV, 1), 0)
    mask = jnp.where(cnt <= _MAXV - 1,
                     (jidx >= cnt).astype(jnp.float32), 0.0)
    vars_ = vars_ * (1.0 - mask)
    crit_ref[0] = crit
    out_ref[0] = outr
    vars_ref[0] = vars_


def _forward_impl(nodes, edges, starter, assignment, cursor_position,
                  vars_in_scope, params):
    del assignment
    nodes2d = nodes.astype(jnp.int32).reshape(_B, _N // 128, 128)
    starter3 = starter.astype(jnp.float32).reshape(_B, _N, 1)
    e3 = edges.astype(jnp.int32).reshape(_B, _E, 3)
    src, dst, ea = e3[:, :, 0], e3[:, :, 1], e3[:, :, 2]
    srcF = jnp.concatenate([src, dst], axis=1)
    dstF = jnp.concatenate([dst, src], axis=1)
    attrF = jnp.concatenate([ea + 1, ea + 7], axis=1)
    src2d = srcF.reshape(_B, _NCH, _ECR, 128)
    dst2d = dstF.reshape(_B, _NCH, _ECR, 128)
    attr2d = attrF.reshape(_B, _NCH, _ECR, 128)
    dstR = dstF.reshape(_B, 1, _EDGES)

    x = pl.pallas_call(
        _embed_body,
        grid=(_B,),
        in_specs=[_spec((1, _N // 128, 128), lambda b: (b, 0, 0)),
                  _spec((1, _N, 1), lambda b: (b, 0, 0)),
                  _spec((_NVOC, _EMBED), lambda b: (0, 0))],
        out_specs=_spec((1, _N, _INS[0]), lambda b: (b, 0, 0)),
        out_shape=jax.ShapeDtypeStruct((_B, _N, _INS[0]), jnp.float32),
    )(nodes2d, starter3, params['node_emb'])

    for i in range(6):
        x = _layer_call(i, x, src2d, dst2d, dstR, attr2d,
                        params['conv%d' % i], params['edge_emb'])

    cur3 = cursor_position.astype(jnp.int32).reshape(_B, 1, 1)
    vis3 = vars_in_scope.astype(jnp.int32).reshape(_B, _MAXV, 1)
    crit, out, vars_ = pl.pallas_call(
        _final_body,
        grid=(_B,),
        in_specs=[_spec((1, _N, _OUTS[5]), lambda b: (b, 0, 0)),
                  _spec((1, 1, 1), lambda b: (b, 0, 0)),
                  _spec((1, _MAXV, 1), lambda b: (b, 0, 0)),
                  _spec((_OUTS[5], 1), lambda b: (0, 0)),
                  _spec((1, 1), lambda b: (0, 0))],
        out_specs=[_spec((1, 1, 1), lambda b: (b, 0, 0)),
                   _spec((1, 1, _OUTS[5]), lambda b: (b, 0, 0)),
                   _spec((1, _MAXV, _OUTS[5]), lambda b: (b, 0, 0))],
        out_shape=[jax.ShapeDtypeStruct((_B, 1, 1), jnp.float32),
                   jax.ShapeDtypeStruct((_B, 1, _OUTS[5]), jnp.float32),
                   jax.ShapeDtypeStruct((_B, _MAXV, _OUTS[5]), jnp.float32)],
    )(x, cur3, vis3, params['critic_W'], params['critic_b'].reshape(1, 1))
    return (crit.reshape(_B, 1), out.reshape(_B, _OUTS[5]), vars_)


kernel = jax.jit(_forward_impl)
